# baseline (speedup 1.0000x reference)
import functools

import jax
import jax.numpy as jnp
from jax import lax
from jax.experimental import pallas as pl
from jax.experimental.pallas import tpu as pltpu
from jax.experimental.pallas import tpu_sc as plsc

VOCAB = 100000
D = 128
W = D // 2
B = 16384
L = 200
BL = B * L

CHUNKS = 4
BLC = BL // CHUNKS
NC = 2
NS = 16
NW = NC * NS
CH = 128
PER_W = BLC // NW
CHUNK_ROWS_PER_W = PER_W // CH
N_GROUPS = CHUNK_ROWS_PER_W // 2
N_PAIR = N_GROUPS // 2


def _sc_gather_body(cw, pw, nw, in_t, out_t, oc, op, on,
                    idx_v, f0, f1, f2, f3, i0, i1, i2, i3,
                    sgf0, swf0, sgf1, swf1, sgi0, swi0, sgi1, swi1):
    wid = lax.axis_index("s") * NC + lax.axis_index("c")
    base_crow = wid * CHUNK_ROWS_PER_W
    fbanks = ((f0, f1, sgf0, swf0), (f2, f3, sgf1, swf1))
    ibanks = ((i0, i1, sgi0, swi0), (i2, i3, sgi1, swi1))

    def drain_writes(bank, out_hbm):
        bufa, bufb, _, sw = bank
        pltpu.make_async_copy(bufa, out_hbm.at[pl.ds(0, CH)], sw).wait()
        pltpu.make_async_copy(bufb, out_hbm.at[pl.ds(0, CH)], sw).wait()

    def fire_gathers(g, bank, table):
        bufa, bufb, sg, _ = bank
        cl = g * 2
        return (pltpu.async_copy(table.at[idx_v.at[cl]], bufa, sg),
                pltpu.async_copy(table.at[idx_v.at[cl + 1]], bufb, sg))

    def fire_writes(g, bank, gh, out_hbm):
        bufa, bufb, _, sw = bank
        row0 = (base_crow + g * 2) * CH
        for h in gh:
            h.wait()
        pltpu.async_copy(bufa, out_hbm.at[pl.ds(row0, CH)], sw)
        pltpu.async_copy(bufb, out_hbm.at[pl.ds(row0 + CH, CH)], sw)

    for idx_hbm, table, out_hbm, banks in (
            (cw, in_t, oc, fbanks), (pw, out_t, op, ibanks), (nw, out_t, on, ibanks)):
        pltpu.sync_copy(idx_hbm.at[pl.ds(base_crow, CHUNK_ROWS_PER_W), :], idx_v)

        def pair_body(i, carry, table=table, out_hbm=out_hbm, banks=banks):
            gh = {}
            for k in range(2):
                bank = banks[k]

                @pl.when(i > 0)
                def _free_bank(bank=bank):
                    drain_writes(bank, out_hbm)

                gh[k] = fire_gathers(2 * i + k, bank, table)
                if k == 1:
                    fire_writes(2 * i, banks[0], gh[0], out_hbm)
            fire_writes(2 * i + 1, banks[1], gh[1], out_hbm)
            return carry

        lax.fori_loop(0, N_PAIR, pair_body, 0)
        for k in range(2):
            drain_writes(banks[k], out_hbm)


_sc_gather = functools.partial(
    pl.kernel,
    mesh=plsc.VectorSubcoreMesh(core_axis_name="c", subcore_axis_name="s"),
    compiler_params=pltpu.CompilerParams(use_tc_tiling_on_sc=False),
    out_type=[jax.ShapeDtypeStruct((BLC, D), jnp.float32),
              jax.ShapeDtypeStruct((BLC, W), jnp.int32),
              jax.ShapeDtypeStruct((BLC, W), jnp.int32)],
    scratch_types=[
        pltpu.VMEM((CHUNK_ROWS_PER_W, CH), jnp.int32),
        pltpu.VMEM((CH, D), jnp.float32),
        pltpu.VMEM((CH, D), jnp.float32),
        pltpu.VMEM((CH, D), jnp.float32),
        pltpu.VMEM((CH, D), jnp.float32),
        pltpu.VMEM((CH, W), jnp.int32),
        pltpu.VMEM((CH, W), jnp.int32),
        pltpu.VMEM((CH, W), jnp.int32),
        pltpu.VMEM((CH, W), jnp.int32),
        pltpu.SemaphoreType.DMA,
        pltpu.SemaphoreType.DMA,
        pltpu.SemaphoreType.DMA,
        pltpu.SemaphoreType.DMA,
        pltpu.SemaphoreType.DMA,
        pltpu.SemaphoreType.DMA,
        pltpu.SemaphoreType.DMA,
        pltpu.SemaphoreType.DMA,
    ],
)(_sc_gather_body)


G = 32
NG = BLC // (G * L)
H = L // 2
LP = 104
L2 = 2 * LP
PAD_TERMS = 2 * L * (L2 - L)
LOG2_F32 = 0.6931471805599453


def _operands(x):
    lo = lax.bitcast_convert_type(lax.shift_left(x, 16), jnp.float32)
    hi = lax.bitcast_convert_type(
        lax.bitwise_and(x, jnp.int32(-65536)), jnp.float32)
    z4 = jnp.zeros((LP - H, W), jnp.float32)
    lo2 = jnp.concatenate([lo[:, :W], z4, lo[:, W:], z4], axis=0)
    hi2 = jnp.concatenate([hi[:, :W], z4, hi[:, W:], z4], axis=0)
    return lo2, hi2


def _tc_loss_body(c_ref, p_ref, n_ref, out_ref):
    g = pl.program_id(0)

    @pl.when(g == 0)
    def _init():
        out_ref[...] = jnp.zeros((1, 1), jnp.float32)

    total = jnp.float32(0.0)
    for b in range(G):
        c = c_ref[b * L:(b + 1) * L, :]
        ce = c[:, :W]
        co = c[:, W:]
        pl_, ph = _operands(p_ref[b * H:(b + 1) * H, :])
        nl, nh = _operands(n_ref[b * H:(b + 1) * H, :])
        dn = (((1,), (1,)), ((), ()))
        ps = (lax.dot_general(ce, pl_, dn, preferred_element_type=jnp.float32)
              + lax.dot_general(co, ph, dn, preferred_element_type=jnp.float32))
        ns = (lax.dot_general(ce, nl, dn, preferred_element_type=jnp.float32)
              + lax.dot_general(co, nh, dn, preferred_element_type=jnp.float32))
        ap = jnp.abs(ps)
        an = jnp.abs(ns)
        lp = jnp.log(1.0 + jnp.exp(-ap))
        ln_ = jnp.log(1.0 + jnp.exp(-an))
        term = (lp + ln_) + ((ap - ps) + (an + ns)) * 0.5
        total = total + jnp.sum(term)
    total = total - jnp.float32(G * PAD_TERMS * LOG2_F32)
    out_ref[...] += jnp.full((1, 1), total, jnp.float32)


def _tc_loss(oc, op, on):
    return pl.pallas_call(
        _tc_loss_body,
        grid=(NG,),
        in_specs=[pl.BlockSpec((G * L, D), lambda i: (i, 0)),
                  pl.BlockSpec((G * H, D), lambda i: (i, 0)),
                  pl.BlockSpec((G * H, D), lambda i: (i, 0))],
        out_specs=pl.BlockSpec((1, 1), lambda i: (0, 0)),
        out_shape=jax.ShapeDtypeStruct((1, 1), jnp.float32),
    )(oc, op, on)


def kernel(center_word, pos_word, neg_word, in_emb, out_emb):
    cw = center_word.reshape(BL // CH, CH)
    pw = pos_word.reshape(BL // CH, CH)
    nw = neg_word.reshape(BL // CH, CH)
    in_shuf = jnp.concatenate([in_emb[:, 0::2], in_emb[:, 1::2]], axis=1)
    out16 = out_emb.astype(jnp.bfloat16)
    ev = lax.bitcast_convert_type(out16[:, 0::2], jnp.uint16).astype(jnp.int32)
    od = lax.bitcast_convert_type(out16[:, 1::2], jnp.uint16).astype(jnp.int32)
    outi = lax.bitwise_or(ev, lax.shift_left(od, 16))
    rows = BLC // CH
    partials = []
    for k in range(CHUNKS):
        sl = slice(k * rows, (k + 1) * rows)
        oc, op, on = _sc_gather(cw[sl], pw[sl], nw[sl], in_shuf, outi)
        partials.append(_tc_loss(oc,
                                 op.reshape(BLC // 2, D),
                                 on.reshape(BLC // 2, D)))
    total = sum(p[0, 0] for p in partials)
    return total * (1.0 / float(BL))


# speedup vs baseline: 1.0463x; 1.0463x over previous
import functools

import jax
import jax.numpy as jnp
from jax import lax
from jax.experimental import pallas as pl
from jax.experimental.pallas import tpu as pltpu
from jax.experimental.pallas import tpu_sc as plsc

VOCAB = 100000
D = 128
W = D // 2
B = 16384
L = 200
BL = B * L

CHUNKS = 4
BLC = BL // CHUNKS
NC = 2
NS = 16
NW = NC * NS
CH = 128
PER_W = BLC // NW
CHUNK_ROWS_PER_W = PER_W // CH
N_GROUPS = CHUNK_ROWS_PER_W // 2
N_PAIR = N_GROUPS // 2


def _make_sc_body(n_streams):
    def body(*args):
        idx_hbms = args[:n_streams]
        table = args[n_streams]
        outs = args[n_streams + 1:2 * n_streams + 1]
        rest = args[2 * n_streams + 1:]
        idx_v = rest[0]
        b0, b1, b2, b3 = rest[1:5]
        sg0, sw0, sg1, sw1 = rest[5:9]
        wid = lax.axis_index("s") * NC + lax.axis_index("c")
        base_crow = wid * CHUNK_ROWS_PER_W
        banks = ((b0, b1, sg0, sw0), (b2, b3, sg1, sw1))

        def drain_writes(bank, out_hbm):
            bufa, bufb, _, sw = bank
            pltpu.make_async_copy(bufa, out_hbm.at[pl.ds(0, CH)], sw).wait()
            pltpu.make_async_copy(bufb, out_hbm.at[pl.ds(0, CH)], sw).wait()

        def fire_gathers(g, bank):
            bufa, bufb, sg, _ = bank
            cl = g * 2
            return (pltpu.async_copy(table.at[idx_v.at[cl]], bufa, sg),
                    pltpu.async_copy(table.at[idx_v.at[cl + 1]], bufb, sg))

        def fire_writes(g, bank, gh, out_hbm):
            bufa, bufb, _, sw = bank
            row0 = (base_crow + g * 2) * CH
            for h in gh:
                h.wait()
            pltpu.async_copy(bufa, out_hbm.at[pl.ds(row0, CH)], sw)
            pltpu.async_copy(bufb, out_hbm.at[pl.ds(row0 + CH, CH)], sw)

        for idx_hbm, out_hbm in zip(idx_hbms, outs):
            pltpu.sync_copy(
                idx_hbm.at[pl.ds(base_crow, CHUNK_ROWS_PER_W), :], idx_v)

            def pair_body(i, carry, out_hbm=out_hbm):
                gh = {}
                for k in range(2):
                    bank = banks[k]

                    @pl.when(i > 0)
                    def _free_bank(bank=bank):
                        drain_writes(bank, out_hbm)

                    gh[k] = fire_gathers(2 * i + k, bank)
                    if k == 1:
                        fire_writes(2 * i, banks[0], gh[0], out_hbm)
                fire_writes(2 * i + 1, banks[1], gh[1], out_hbm)
                return carry

            lax.fori_loop(0, N_PAIR, pair_body, 0)
            for k in range(2):
                drain_writes(banks[k], out_hbm)
    return body


_MESH = plsc.VectorSubcoreMesh(core_axis_name="c", subcore_axis_name="s")
_SEMS = [pltpu.SemaphoreType.DMA] * 4

_sc_gather_f32 = functools.partial(
    pl.kernel,
    mesh=_MESH,
    out_type=[jax.ShapeDtypeStruct((BLC, D), jnp.float32)],
    scratch_types=[pltpu.VMEM((CHUNK_ROWS_PER_W, CH), jnp.int32)]
    + [pltpu.VMEM((CH, D), jnp.float32)] * 4 + _SEMS,
)(_make_sc_body(1))

_sc_gather_i32 = functools.partial(
    pl.kernel,
    mesh=_MESH,
    compiler_params=pltpu.CompilerParams(use_tc_tiling_on_sc=False),
    out_type=[jax.ShapeDtypeStruct((BLC, W), jnp.int32)] * 2,
    scratch_types=[pltpu.VMEM((CHUNK_ROWS_PER_W, CH), jnp.int32)]
    + [pltpu.VMEM((CH, W), jnp.int32)] * 4 + _SEMS,
)(_make_sc_body(2))


G = 32
NG = BLC // (G * L)
H = L // 2
LP = 104
L2 = 2 * LP
PAD_TERMS = 2 * L * (L2 - L)
LOG2_F32 = 0.6931471805599453


def _operands(x):
    lo = lax.bitcast_convert_type(lax.shift_left(x, 16), jnp.float32)
    hi = lax.bitcast_convert_type(
        lax.bitwise_and(x, jnp.int32(-65536)), jnp.float32)
    z4 = jnp.zeros((LP - H, W), jnp.float32)
    lo2 = jnp.concatenate([lo[:, :W], z4, lo[:, W:], z4], axis=0)
    hi2 = jnp.concatenate([hi[:, :W], z4, hi[:, W:], z4], axis=0)
    return lo2, hi2


def _tc_loss_body(c_ref, p_ref, n_ref, out_ref):
    g = pl.program_id(0)

    @pl.when(g == 0)
    def _init():
        out_ref[...] = jnp.zeros((1, 1), jnp.float32)

    total = jnp.float32(0.0)
    for b in range(G):
        c = c_ref[b * L:(b + 1) * L, :]
        ce = c[:, :W]
        co = c[:, W:]
        pl_, ph = _operands(p_ref[b * H:(b + 1) * H, :])
        nl, nh = _operands(n_ref[b * H:(b + 1) * H, :])
        dn = (((1,), (1,)), ((), ()))
        ps = (lax.dot_general(ce, pl_, dn, preferred_element_type=jnp.float32)
              + lax.dot_general(co, ph, dn, preferred_element_type=jnp.float32))
        ns = (lax.dot_general(ce, nl, dn, preferred_element_type=jnp.float32)
              + lax.dot_general(co, nh, dn, preferred_element_type=jnp.float32))
        ap = jnp.abs(ps)
        an = jnp.abs(ns)
        lp = jnp.log(1.0 + jnp.exp(-ap))
        ln_ = jnp.log(1.0 + jnp.exp(-an))
        term = (lp + ln_) + ((ap - ps) + (an + ns)) * 0.5
        total = total + jnp.sum(term)
    total = total - jnp.float32(G * PAD_TERMS * LOG2_F32)
    out_ref[...] += jnp.full((1, 1), total, jnp.float32)


def _tc_loss(oc, op, on):
    return pl.pallas_call(
        _tc_loss_body,
        grid=(NG,),
        in_specs=[pl.BlockSpec((G * L, D), lambda i: (i, 0)),
                  pl.BlockSpec((G * H, D), lambda i: (i, 0)),
                  pl.BlockSpec((G * H, D), lambda i: (i, 0))],
        out_specs=pl.BlockSpec((1, 1), lambda i: (0, 0)),
        out_shape=jax.ShapeDtypeStruct((1, 1), jnp.float32),
    )(oc, op, on)


def kernel(center_word, pos_word, neg_word, in_emb, out_emb):
    cw = center_word.reshape(BL // CH, CH)
    pw = pos_word.reshape(BL // CH, CH)
    nw = neg_word.reshape(BL // CH, CH)
    in_shuf = jnp.concatenate([in_emb[:, 0::2], in_emb[:, 1::2]], axis=1)
    out16 = out_emb.astype(jnp.bfloat16)
    ev = lax.bitcast_convert_type(out16[:, 0::2], jnp.uint16).astype(jnp.int32)
    od = lax.bitcast_convert_type(out16[:, 1::2], jnp.uint16).astype(jnp.int32)
    outi = lax.bitwise_or(ev, lax.shift_left(od, 16))
    rows = BLC // CH
    partials = []
    for k in range(CHUNKS):
        sl = slice(k * rows, (k + 1) * rows)
        (oc,) = _sc_gather_f32(cw[sl], in_shuf)
        op, on = _sc_gather_i32(pw[sl], nw[sl], outi)
        partials.append(_tc_loss(oc,
                                 op.reshape(BLC // 2, D),
                                 on.reshape(BLC // 2, D)))
    total = sum(p[0, 0] for p in partials)
    return total * (1.0 / float(BL))


# speedup vs baseline: 1.2993x; 1.2418x over previous
import functools

import jax
import jax.numpy as jnp
from jax import lax
from jax.experimental import pallas as pl
from jax.experimental.pallas import tpu as pltpu
from jax.experimental.pallas import tpu_sc as plsc

VOCAB = 100000
D = 128
B = 16384
L = 200
BL = B * L

CHUNKS = 4
BLC = BL // CHUNKS
NC = 2
NS = 16
NW = NC * NS
CH = 128
SUP = 8
PER_W = BLC // NW
N_SUP = PER_W // (CH * SUP)
CHUNK_ROWS_PER_W = PER_W // CH


N_GROUPS = CHUNK_ROWS_PER_W // 2
N_TRI = N_GROUPS // 3
REM = N_GROUPS - 3 * N_TRI


def _sc_gather_body(cw, pw, nw, in_t, out_t, oc, op, on,
                    idx_v, b0, b1, b2, b3, b4, b5,
                    sg0, sg1, sg2, sw0, sw1, sw2):
    wid = lax.axis_index("s") * NC + lax.axis_index("c")
    base_crow = wid * CHUNK_ROWS_PER_W
    banks = ((b0, b1, sg0, sw0), (b2, b3, sg1, sw1), (b4, b5, sg2, sw2))

    def drain_writes(bank, out_hbm):
        bufa, bufb, _, sw = bank
        pltpu.make_async_copy(bufa, out_hbm.at[pl.ds(0, CH)], sw).wait()
        pltpu.make_async_copy(bufb, out_hbm.at[pl.ds(0, CH)], sw).wait()

    def fire_gathers(g, bank, table):
        bufa, bufb, sg, _ = bank
        cl = g * 2
        return (pltpu.async_copy(table.at[idx_v.at[cl]], bufa, sg),
                pltpu.async_copy(table.at[idx_v.at[cl + 1]], bufb, sg))

    def fire_writes(g, bank, gh, out_hbm):
        bufa, bufb, _, sw = bank
        row0 = (base_crow + g * 2) * CH
        for h in gh:
            h.wait()
        pltpu.async_copy(bufa, out_hbm.at[pl.ds(row0, CH)], sw)
        pltpu.async_copy(bufb, out_hbm.at[pl.ds(row0 + CH, CH)], sw)

    for idx_hbm, table, out_hbm in ((cw, in_t, oc), (pw, out_t, op), (nw, out_t, on)):
        pltpu.sync_copy(idx_hbm.at[pl.ds(base_crow, CHUNK_ROWS_PER_W), :], idx_v)

        def tri_body(i, carry, table=table, out_hbm=out_hbm):
            gh = {}
            for k in range(3):
                bank = banks[k]

                @pl.when(i > 0)
                def _free_bank(bank=bank):
                    drain_writes(bank, out_hbm)

                gh[k] = fire_gathers(3 * i + k, bank, table)
                if k >= 1:
                    fire_writes(3 * i + k - 1, banks[k - 1], gh[k - 1], out_hbm)
            fire_writes(3 * i + 2, banks[2], gh[2], out_hbm)
            return carry

        lax.fori_loop(0, N_TRI, tri_body, 0)
        for r in range(REM):
            g = 3 * N_TRI + r
            bank = banks[r]
            drain_writes(bank, out_hbm)
            gh = fire_gathers(g, bank, table)
            fire_writes(g, bank, gh, out_hbm)
        for k in range(3):
            drain_writes(banks[k], out_hbm)


_sc_gather = functools.partial(
    pl.kernel,
    mesh=plsc.VectorSubcoreMesh(core_axis_name="c", subcore_axis_name="s"),
    out_type=[jax.ShapeDtypeStruct((BLC, D), jnp.float32)] * 3,
    scratch_types=[
        pltpu.VMEM((CHUNK_ROWS_PER_W, CH), jnp.int32),
        pltpu.VMEM((CH, D), jnp.float32),
        pltpu.VMEM((CH, D), jnp.float32),
        pltpu.VMEM((CH, D), jnp.float32),
        pltpu.VMEM((CH, D), jnp.float32),
        pltpu.VMEM((CH, D), jnp.float32),
        pltpu.VMEM((CH, D), jnp.float32),
        pltpu.SemaphoreType.DMA,
        pltpu.SemaphoreType.DMA,
        pltpu.SemaphoreType.DMA,
        pltpu.SemaphoreType.DMA,
        pltpu.SemaphoreType.DMA,
        pltpu.SemaphoreType.DMA,
    ],
)(_sc_gather_body)


G = 32
NG = BLC // (G * L)

LOG2E = 1.4426950408889634
LN2 = 0.6931471805599453
INV = 0.5 / LN2


def _tc_loss_body(c_ref, p_ref, n_ref, out_ref):
    g = pl.program_id(0)

    @pl.when(g == 0)
    def _init():
        out_ref[...] = jnp.zeros((1, 1), jnp.float32)

    total = jnp.float32(0.0)
    for b in range(G):
        c = c_ref[b * L:(b + 1) * L, :]
        p = p_ref[b * L:(b + 1) * L, :]
        n = n_ref[b * L:(b + 1) * L, :]
        dn = (((1,), (1,)), ((), ()))
        ps = lax.dot_general(c, p, dn, preferred_element_type=jnp.float32)
        ns = lax.dot_general(c, n, dn, preferred_element_type=jnp.float32)
        ap = jnp.abs(ps)
        an = jnp.abs(ns)
        lp = jnp.log(1.0 + jnp.exp(-ap))
        ln_ = jnp.log(1.0 + jnp.exp(-an))
        term = (lp + ln_) + ((ap - ps) + (an + ns)) * 0.5
        total = total + jnp.sum(term)
    out_ref[...] += jnp.full((1, 1), total, jnp.float32)


def _tc_loss(oc, op, on):
    return pl.pallas_call(
        _tc_loss_body,
        grid=(NG,),
        in_specs=[pl.BlockSpec((G * L, D), lambda i: (i, 0))] * 3,
        out_specs=pl.BlockSpec((1, 1), lambda i: (0, 0)),
        out_shape=jax.ShapeDtypeStruct((1, 1), jnp.float32),
    )(oc, op, on)


def kernel(center_word, pos_word, neg_word, in_emb, out_emb):
    cw = center_word.reshape(BL // CH, CH)
    pw = pos_word.reshape(BL // CH, CH)
    nw = neg_word.reshape(BL // CH, CH)
    rows = BLC // CH
    partials = []
    for k in range(CHUNKS):
        sl = slice(k * rows, (k + 1) * rows)
        oc, op, on = _sc_gather(cw[sl], pw[sl], nw[sl], in_emb, out_emb)
        partials.append(_tc_loss(oc, op, on))
    total = sum(p[0, 0] for p in partials)
    return total * (1.0 / float(BL))
